# Initial kernel scaffold; baseline (speedup 1.0000x reference)
import jax
import jax.numpy as jnp
from jax.experimental import pallas as pl


def kernel(x, rows, cols, vals, mask):
    raise NotImplementedError("write your pallas kernel here")


# speedup vs baseline: 16.9085x; 16.9085x over previous
import functools

import jax
import jax.numpy as jnp
from jax import lax
from jax.experimental import pallas as pl
from jax.experimental.pallas import tpu as pltpu
from jax.experimental.pallas import tpu_sc as plsc

B = 16
C = 3
H = 256
W = 256
OUT_H = 128
OUT_W = 64
NNZ = 98304
OUT_DIM = OUT_H * OUT_W * C
IN_DIM = H * W * C
L = 16

CHUNK = 12288


def _divmod3(v):
    vf = v.astype(jnp.float32)
    q = (vf * jnp.float32(1.0 / 3.0) + jnp.float32(0.5)).astype(jnp.int32)
    m = v - q * 3
    neg = m < 0
    q = jnp.where(neg, q - 1, q)
    m = jnp.where(neg, m + 3, m)
    return q, m


def _sc_spmv(x1d, rows, cols, vals):
    info = plsc.get_sparse_core_info()
    num_cores, num_subcores = info.num_cores, info.num_subcores
    items_per_core = B // num_cores
    tiles_per_item = num_subcores // items_per_core
    nnz_per_tile = NNZ // tiles_per_item
    n_chunks = nnz_per_tile // CHUNK
    out_slice = OUT_DIM // tiles_per_item

    mesh = plsc.VectorSubcoreMesh(core_axis_name="c", subcore_axis_name="s")

    @functools.partial(
        pl.kernel,
        out_type=jax.ShapeDtypeStruct((B * OUT_DIM,), jnp.float32),
        mesh=mesh,
        scratch_types=[
            pltpu.VMEM_SHARED((items_per_core * OUT_DIM,), jnp.float32),
            pltpu.VMEM((CHUNK,), jnp.int32),
            pltpu.VMEM((CHUNK,), jnp.int32),
            pltpu.VMEM((CHUNK,), jnp.float32),
            pltpu.VMEM((CHUNK,), jnp.int32),
            pltpu.VMEM((CHUNK,), jnp.int32),
            pltpu.VMEM((CHUNK,), jnp.float32),
            pltpu.SemaphoreType.DMA,
        ],
    )
    def run(x_hbm, rows_hbm, cols_hbm, vals_hbm, out_hbm,
            shared_acc, cols_v, rows_v, vals_v,
            gidx_v, sidx_v, xv_v, sem):
        cid = lax.axis_index("c")
        sid = lax.axis_index("s")
        slot = sid // tiles_per_item
        half = sid % tiles_per_item
        item = cid * items_per_core + slot

        def zero_body(i, _):
            xv_v[pl.ds(i * L, L)] = jnp.zeros((L,), jnp.float32)
            return 0
        lax.fori_loop(0, CHUNK // L, zero_body, 0)
        pltpu.sync_copy(
            xv_v.at[pl.ds(0, out_slice)],
            shared_acc.at[pl.ds(slot * OUT_DIM + half * out_slice, out_slice)])

        plsc.subcore_barrier()

        gbase = item * IN_DIM
        sbase = slot * OUT_DIM
        nnz_base = half * nnz_per_tile

        def chunk_body(ci, _):
            base = nnz_base + ci * CHUNK
            pltpu.sync_copy(cols_hbm.at[item, pl.ds(base, CHUNK)], cols_v)
            pltpu.sync_copy(rows_hbm.at[item, pl.ds(base, CHUNK)], rows_v)
            pltpu.sync_copy(vals_hbm.at[item, pl.ds(base, CHUNK)], vals_v)

            def idx_body(i, _):
                sl = pl.ds(i * L, L)
                cc = cols_v[sl]
                q, m = _divmod3(cc)
                gidx_v[sl] = m * (H * W) + q + gbase
                rr = rows_v[sl]
                q2, m2 = _divmod3(rr)
                sidx_v[sl] = m2 * (OUT_H * OUT_W) + q2 + sbase
                return 0
            lax.fori_loop(0, CHUNK // L, idx_body, 0)

            pltpu.async_copy(x_hbm.at[gidx_v], xv_v, sem).wait()

            def mul_body(i, _):
                sl = pl.ds(i * L, L)
                xv_v[sl] = xv_v[sl] * vals_v[sl]
                return 0
            lax.fori_loop(0, CHUNK // L, mul_body, 0)

            pltpu.sync_copy(xv_v, shared_acc.at[sidx_v], add=True)
            return 0
        lax.fori_loop(0, n_chunks, chunk_body, 0)

        plsc.subcore_barrier()

        pltpu.sync_copy(
            shared_acc.at[pl.ds(slot * OUT_DIM + half * out_slice, out_slice)],
            out_hbm.at[pl.ds(item * OUT_DIM + half * out_slice, out_slice)])

    return run(x1d, rows, cols, vals)


def kernel(x, rows, cols, vals, mask):
    x1d = x.reshape(B * IN_DIM)
    out = _sc_spmv(x1d, rows, cols, vals)
    result = out.reshape(B, C, OUT_H, OUT_W)
    masks = jnp.transpose(mask, (0, 3, 1, 2))
    return (result, masks)
